# baseline (speedup 1.0000x reference)
import jax
import jax.numpy as jnp
from jax import lax
from jax.experimental import pallas as pl
from jax.experimental.pallas import tpu as pltpu


def _conv(x, w, b, stride):
    y = lax.conv_general_dilated(x, w, (stride, stride), ((1, 1), (1, 1)),
                                 dimension_numbers=('NCHW', 'OIHW', 'NCHW'))
    return y + b[None, :, None, None]


def _conv_t(x, w, b):
    y = lax.conv_transpose(x, w, (2, 2), ((1, 1), (1, 1)),
                           dimension_numbers=('NCHW', 'OIHW', 'NCHW'))
    return y + b[None, :, None, None]


def _instance_norm(x, eps=1e-5):
    m = jnp.mean(x, axis=(2, 3), keepdims=True)
    v = jnp.var(x, axis=(2, 3), keepdims=True)
    return (x - m) / jnp.sqrt(v + eps)


def _batch_norm(x, gamma, beta, eps=1e-5):
    m = jnp.mean(x, axis=(0, 2, 3), keepdims=True)
    v = jnp.var(x, axis=(0, 2, 3), keepdims=True)
    xn = (x - m) / jnp.sqrt(v + eps)
    return xn * gamma[None, :, None, None] + beta[None, :, None, None]


def _rvq_body(z_ref, cb_ref, q_ref, loss_ref):
    z = z_ref[...]
    n_levels = cb_ref.shape[0]
    num_k = cb_ref.shape[1]
    c = z.shape[1]
    resid = z
    qsum = jnp.zeros_like(z)
    loss = jnp.zeros((z.shape[0], 1), jnp.float32)
    iota_k = lax.broadcasted_iota(jnp.int32, (1, num_k), 1)
    for lvl in range(n_levels):
        cbl = cb_ref[lvl]
        m = jnp.dot(resid.astype(jnp.bfloat16), cbl.T.astype(jnp.bfloat16),
                    preferred_element_type=jnp.float32)
        d = ((jnp.sum(resid * resid, axis=1, keepdims=True) - 2.0 * m)
             + jnp.sum(cbl * cbl, axis=1)[None, :])
        idx = jnp.argmin(d, axis=1)
        onehot = (idx[:, None] == iota_k).astype(jnp.float32)
        q = jnp.dot(onehot, cbl, preferred_element_type=jnp.float32,
                    precision=lax.Precision.HIGHEST)
        loss = loss + jnp.sum((q - resid) ** 2, axis=1, keepdims=True)
        resid = resid - q
        qsum = qsum + q
    q_ref[...] = qsum
    loss_ref[...] = loss


def _rvq(zf, cbs, blk):
    n, c = zf.shape
    blk = min(blk, n)
    n_levels, num_k, _ = cbs.shape
    nblk = n // blk

    const = lambda i: (0, 0, 0)
    qf, losses = pl.pallas_call(
        _rvq_body,
        grid=(nblk,),
        in_specs=[
            pl.BlockSpec((blk, c), lambda i: (i, 0)),
            pl.BlockSpec((n_levels, num_k, c), const),
        ],
        out_specs=[
            pl.BlockSpec((blk, c), lambda i: (i, 0)),
            pl.BlockSpec((blk, 1), lambda i: (i, 0)),
        ],
        out_shape=[
            jax.ShapeDtypeStruct((n, c), jnp.float32),
            jax.ShapeDtypeStruct((n, 1), jnp.float32),
        ],
    )(zf, cbs)
    return qf, losses


def kernel(x, enc_w1, enc_b1, enc_w2, enc_b2, codebooks,
           dec_w1, dec_b1, bn_gamma, bn_beta, dec_w2, dec_b2):
    h = _conv(x, enc_w1, enc_b1, 2)
    h = jax.nn.relu(_instance_norm(h))
    h = _conv(h, enc_w2, enc_b2, 2)
    z = jax.nn.relu(_instance_norm(h))

    b, c, hh, ww = z.shape
    zf = jnp.transpose(z, (0, 2, 3, 1)).reshape(-1, c)
    cbs = codebooks.reshape(-1, codebooks.shape[-2], codebooks.shape[-1])

    qf, losses = _rvq(zf, cbs, blk=128)
    n = zf.shape[0]
    total_loss = (jnp.sum(losses) * (1.25 / (n * c))).astype(jnp.float32)

    qz = jnp.transpose(qf.reshape(b, hh, ww, c), (0, 3, 1, 2))
    d = _conv_t(qz, dec_w1, dec_b1)
    d = _batch_norm(d, bn_gamma, bn_beta)
    out = _conv_t(d, dec_w2, dec_b2)
    return out, total_loss


# speedup vs baseline: 1.1996x; 1.1996x over previous
import jax
import jax.numpy as jnp
from jax import lax
from jax.experimental import pallas as pl
from jax.experimental.pallas import tpu as pltpu


def _conv(x, w, b, stride):
    y = lax.conv_general_dilated(x, w, (stride, stride), ((1, 1), (1, 1)),
                                 dimension_numbers=('NCHW', 'OIHW', 'NCHW'))
    return y + b[None, :, None, None]


def _conv_t(x, w, b):
    y = lax.conv_transpose(x, w, (2, 2), ((1, 1), (1, 1)),
                           dimension_numbers=('NCHW', 'OIHW', 'NCHW'))
    return y + b[None, :, None, None]


def _instance_norm(x, eps=1e-5):
    m = jnp.mean(x, axis=(2, 3), keepdims=True)
    v = jnp.var(x, axis=(2, 3), keepdims=True)
    return (x - m) / jnp.sqrt(v + eps)


def _batch_norm(x, gamma, beta, eps=1e-5):
    m = jnp.mean(x, axis=(0, 2, 3), keepdims=True)
    v = jnp.var(x, axis=(0, 2, 3), keepdims=True)
    xn = (x - m) / jnp.sqrt(v + eps)
    return xn * gamma[None, :, None, None] + beta[None, :, None, None]


def _rvq_body(z_ref, cb_ref, q_ref, loss_ref):
    z = z_ref[...]
    n_levels = cb_ref.shape[0]
    num_k = cb_ref.shape[1]
    c = z.shape[1]
    resid = z
    qsum = jnp.zeros_like(z)
    loss = jnp.zeros((z.shape[0], 1), jnp.float32)
    iota_k = lax.broadcasted_iota(jnp.int32, (1, num_k), 1)
    for lvl in range(n_levels):
        cbl = cb_ref[lvl]
        m = jnp.dot(resid.astype(jnp.bfloat16), cbl.T.astype(jnp.bfloat16),
                    preferred_element_type=jnp.float32)
        d = ((jnp.sum(resid * resid, axis=1, keepdims=True) - 2.0 * m)
             + jnp.sum(cbl * cbl, axis=1)[None, :])
        idx = jnp.argmin(d, axis=1)
        onehot = (idx[:, None] == iota_k).astype(jnp.float32)
        q = jnp.dot(onehot, cbl, preferred_element_type=jnp.float32,
                    precision=lax.Precision.HIGHEST)
        loss = loss + jnp.sum((q - resid) ** 2, axis=1, keepdims=True)
        resid = resid - q
        qsum = qsum + q
    q_ref[...] = qsum
    loss_ref[...] = loss


def _rvq(zf, cbs, blk):
    n, c = zf.shape
    blk = min(blk, n)
    n_levels, num_k, _ = cbs.shape
    nblk = n // blk

    const = lambda i: (0, 0, 0)
    qf, losses = pl.pallas_call(
        _rvq_body,
        grid=(nblk,),
        in_specs=[
            pl.BlockSpec((blk, c), lambda i: (i, 0)),
            pl.BlockSpec((n_levels, num_k, c), const),
        ],
        out_specs=[
            pl.BlockSpec((blk, c), lambda i: (i, 0)),
            pl.BlockSpec((blk, 1), lambda i: (i, 0)),
        ],
        out_shape=[
            jax.ShapeDtypeStruct((n, c), jnp.float32),
            jax.ShapeDtypeStruct((n, 1), jnp.float32),
        ],
    )(zf, cbs)
    return qf, losses


def kernel(x, enc_w1, enc_b1, enc_w2, enc_b2, codebooks,
           dec_w1, dec_b1, bn_gamma, bn_beta, dec_w2, dec_b2):
    h = _conv(x, enc_w1, enc_b1, 2)
    h = jax.nn.relu(_instance_norm(h))
    h = _conv(h, enc_w2, enc_b2, 2)
    z = jax.nn.relu(_instance_norm(h))

    b, c, hh, ww = z.shape
    zf = jnp.transpose(z, (0, 2, 3, 1)).reshape(-1, c)
    cbs = codebooks.reshape(-1, codebooks.shape[-2], codebooks.shape[-1])

    qf, losses = _rvq(zf, cbs, blk=256)
    n = zf.shape[0]
    total_loss = (jnp.sum(losses) * (1.25 / (n * c))).astype(jnp.float32)

    qz = jnp.transpose(qf.reshape(b, hh, ww, c), (0, 3, 1, 2))
    d = _conv_t(qz, dec_w1, dec_b1)
    d = _batch_norm(d, bn_gamma, bn_beta)
    out = _conv_t(d, dec_w2, dec_b2)
    return out, total_loss


# speedup vs baseline: 1.2811x; 1.0679x over previous
import jax
import jax.numpy as jnp
from jax import lax
from jax.experimental import pallas as pl
from jax.experimental.pallas import tpu as pltpu


def _conv(x, w, b, stride):
    y = lax.conv_general_dilated(x, w, (stride, stride), ((1, 1), (1, 1)),
                                 dimension_numbers=('NCHW', 'OIHW', 'NCHW'))
    return y + b[None, :, None, None]


def _conv_t(x, w, b):
    y = lax.conv_transpose(x, w, (2, 2), ((1, 1), (1, 1)),
                           dimension_numbers=('NCHW', 'OIHW', 'NCHW'))
    return y + b[None, :, None, None]


def _instance_norm(x, eps=1e-5):
    m = jnp.mean(x, axis=(2, 3), keepdims=True)
    v = jnp.var(x, axis=(2, 3), keepdims=True)
    return (x - m) / jnp.sqrt(v + eps)


def _batch_norm(x, gamma, beta, eps=1e-5):
    m = jnp.mean(x, axis=(0, 2, 3), keepdims=True)
    v = jnp.var(x, axis=(0, 2, 3), keepdims=True)
    xn = (x - m) / jnp.sqrt(v + eps)
    return xn * gamma[None, :, None, None] + beta[None, :, None, None]


def _rvq_body(z_ref, cb_ref, q_ref, loss_ref):
    z = z_ref[...]
    n_levels = cb_ref.shape[0]
    num_k = cb_ref.shape[1]
    c = z.shape[1]
    resid = z
    qsum = jnp.zeros_like(z)
    loss = jnp.zeros((z.shape[0], 1), jnp.float32)
    iota_k = lax.broadcasted_iota(jnp.int32, (1, num_k), 1)
    for lvl in range(n_levels):
        cbl = cb_ref[lvl]
        m = jnp.dot(resid.astype(jnp.bfloat16), cbl.T.astype(jnp.bfloat16),
                    preferred_element_type=jnp.float32)
        d = ((jnp.sum(resid * resid, axis=1, keepdims=True) - 2.0 * m)
             + jnp.sum(cbl * cbl, axis=1)[None, :])
        idx = jnp.argmin(d, axis=1)
        onehot = (idx[:, None] == iota_k).astype(jnp.float32)
        q = jnp.dot(onehot, cbl, preferred_element_type=jnp.float32,
                    precision=lax.Precision.HIGHEST)
        loss = loss + jnp.sum((q - resid) ** 2, axis=1, keepdims=True)
        resid = resid - q
        qsum = qsum + q
    q_ref[...] = qsum
    loss_ref[...] = loss


def _rvq(zf, cbs, blk):
    n, c = zf.shape
    blk = min(blk, n)
    n_levels, num_k, _ = cbs.shape
    nblk = n // blk

    const = lambda i: (0, 0, 0)
    qf, losses = pl.pallas_call(
        _rvq_body,
        grid=(nblk,),
        in_specs=[
            pl.BlockSpec((blk, c), lambda i: (i, 0)),
            pl.BlockSpec((n_levels, num_k, c), const),
        ],
        out_specs=[
            pl.BlockSpec((blk, c), lambda i: (i, 0)),
            pl.BlockSpec((blk, 1), lambda i: (i, 0)),
        ],
        out_shape=[
            jax.ShapeDtypeStruct((n, c), jnp.float32),
            jax.ShapeDtypeStruct((n, 1), jnp.float32),
        ],
        compiler_params=pltpu.CompilerParams(
            dimension_semantics=("arbitrary",),
            vmem_limit_bytes=128 * 1024 * 1024,
        ),
    )(zf, cbs)
    return qf, losses


def kernel(x, enc_w1, enc_b1, enc_w2, enc_b2, codebooks,
           dec_w1, dec_b1, bn_gamma, bn_beta, dec_w2, dec_b2):
    h = _conv(x, enc_w1, enc_b1, 2)
    h = jax.nn.relu(_instance_norm(h))
    h = _conv(h, enc_w2, enc_b2, 2)
    z = jax.nn.relu(_instance_norm(h))

    b, c, hh, ww = z.shape
    zf = jnp.transpose(z, (0, 2, 3, 1)).reshape(-1, c)
    cbs = codebooks.reshape(-1, codebooks.shape[-2], codebooks.shape[-1])

    qf, losses = _rvq(zf, cbs, blk=512)
    n = zf.shape[0]
    total_loss = (jnp.sum(losses) * (1.25 / (n * c))).astype(jnp.float32)

    qz = jnp.transpose(qf.reshape(b, hh, ww, c), (0, 3, 1, 2))
    d = _conv_t(qz, dec_w1, dec_b1)
    d = _batch_norm(d, bn_gamma, bn_beta)
    out = _conv_t(d, dec_w2, dec_b2)
    return out, total_loss


# speedup vs baseline: 1.3300x; 1.0382x over previous
import jax
import jax.numpy as jnp
from jax import lax
from jax.experimental import pallas as pl
from jax.experimental.pallas import tpu as pltpu


def _conv(x, w, b, stride):
    y = lax.conv_general_dilated(x, w, (stride, stride), ((1, 1), (1, 1)),
                                 dimension_numbers=('NCHW', 'OIHW', 'NCHW'))
    return y + b[None, :, None, None]


def _conv_t(x, w, b):
    y = lax.conv_transpose(x, w, (2, 2), ((1, 1), (1, 1)),
                           dimension_numbers=('NCHW', 'OIHW', 'NCHW'))
    return y + b[None, :, None, None]


def _instance_norm(x, eps=1e-5):
    m = jnp.mean(x, axis=(2, 3), keepdims=True)
    v = jnp.var(x, axis=(2, 3), keepdims=True)
    return (x - m) / jnp.sqrt(v + eps)


def _batch_norm(x, gamma, beta, eps=1e-5):
    m = jnp.mean(x, axis=(0, 2, 3), keepdims=True)
    v = jnp.var(x, axis=(0, 2, 3), keepdims=True)
    xn = (x - m) / jnp.sqrt(v + eps)
    return xn * gamma[None, :, None, None] + beta[None, :, None, None]


def _rvq_body(z_ref, cb_ref, q_ref, loss_ref):
    z = z_ref[...]
    n_levels = cb_ref.shape[0]
    num_k = cb_ref.shape[1]
    c = z.shape[1]
    resid = z
    qsum = jnp.zeros_like(z)
    loss = jnp.zeros((z.shape[0], 1), jnp.float32)
    iota_k = lax.broadcasted_iota(jnp.int32, (1, num_k), 1)
    for lvl in range(n_levels):
        cbl = cb_ref[lvl]
        m = jnp.dot(resid.astype(jnp.bfloat16), cbl.T.astype(jnp.bfloat16),
                    preferred_element_type=jnp.float32)
        d = ((jnp.sum(resid * resid, axis=1, keepdims=True) - 2.0 * m)
             + jnp.sum(cbl * cbl, axis=1)[None, :])
        idx = jnp.argmin(d, axis=1)
        onehot = (idx[:, None] == iota_k).astype(jnp.float32)
        q = jnp.dot(onehot, cbl, preferred_element_type=jnp.float32,
                    precision=lax.Precision.HIGHEST)
        loss = loss + jnp.sum((q - resid) ** 2, axis=1, keepdims=True)
        resid = resid - q
        qsum = qsum + q
    q_ref[...] = qsum
    loss_ref[...] = loss


def _rvq(zf, cbs, blk):
    n, c = zf.shape
    blk = min(blk, n)
    n_levels, num_k, _ = cbs.shape
    nblk = n // blk

    const = lambda i: (0, 0, 0)
    qf, losses = pl.pallas_call(
        _rvq_body,
        grid=(nblk,),
        in_specs=[
            pl.BlockSpec((blk, c), lambda i: (i, 0)),
            pl.BlockSpec((n_levels, num_k, c), const),
        ],
        out_specs=[
            pl.BlockSpec((blk, c), lambda i: (i, 0)),
            pl.BlockSpec((blk, 1), lambda i: (i, 0)),
        ],
        out_shape=[
            jax.ShapeDtypeStruct((n, c), jnp.float32),
            jax.ShapeDtypeStruct((n, 1), jnp.float32),
        ],
        compiler_params=pltpu.CompilerParams(
            dimension_semantics=("arbitrary",),
            vmem_limit_bytes=128 * 1024 * 1024,
        ),
    )(zf, cbs)
    return qf, losses


def kernel(x, enc_w1, enc_b1, enc_w2, enc_b2, codebooks,
           dec_w1, dec_b1, bn_gamma, bn_beta, dec_w2, dec_b2):
    h = _conv(x, enc_w1, enc_b1, 2)
    h = jax.nn.relu(_instance_norm(h))
    h = _conv(h, enc_w2, enc_b2, 2)
    z = jax.nn.relu(_instance_norm(h))

    b, c, hh, ww = z.shape
    zf = jnp.transpose(z, (0, 2, 3, 1)).reshape(-1, c)
    cbs = codebooks.reshape(-1, codebooks.shape[-2], codebooks.shape[-1])

    qf, losses = _rvq(zf, cbs, blk=896)
    n = zf.shape[0]
    total_loss = (jnp.sum(losses) * (1.25 / (n * c))).astype(jnp.float32)

    qz = jnp.transpose(qf.reshape(b, hh, ww, c), (0, 3, 1, 2))
    d = _conv_t(qz, dec_w1, dec_b1)
    d = _batch_norm(d, bn_gamma, bn_beta)
    out = _conv_t(d, dec_w2, dec_b2)
    return out, total_loss
